# baseline (speedup 1.0000x reference)
import functools

import jax
import jax.numpy as jnp
import numpy as np
from jax import lax
from jax.experimental import pallas as pl
from jax.experimental.pallas import tpu as pltpu
from jax.experimental.pallas import tpu_sc as plsc

_FIELD_DIMS = [100000] * 26
_F = 26
_ED = 16
_B = 4096
_E = 8
_T = 2
_EMB_OUT = (_F + 1) * _ED

_NW = 32
_PER_W = _B * _F // _NW
_CH = 128
_NCH = _PER_W // _CH
_NPAT = 13
_OUTR = _PER_W * _ED // 128

_BB = 512


def _offs_pattern():
    offsets = np.concatenate([[0], np.cumsum(_FIELD_DIMS)[:-1]]).astype(np.int32)
    pat = np.array([offsets[q % _F] for q in range(_NPAT * 16)], dtype=np.int32)
    return pat.reshape(_NPAT, 16)


_NLINE = 325000
_NCHUNK = 2539
_TAILC = 64


def _sc_relayout_body(tt_hbm, tail_hbm, out_hbm, slab0, slab1, outb0, outb1,
                      sem0, sem1, semo0, semo1):
    wid = lax.axis_index("s") * 2 + lax.axis_index("c")
    iota16 = lax.iota(jnp.int32, 16)
    rowvecs = [iota16 + 16 * jj for jj in range(8)]
    colvecs = [jnp.full((16,), c, jnp.int32) for c in range(128)]

    def fire_in(m, slab, sem):
        c0 = m * 1024
        for dt in range(2):
            for j in range(8):
                pltpu.async_copy(
                    tt_hbm.at[pl.ds(dt * 8, 8), pl.ds(c0 + j * 128, 128)],
                    slab.at[pl.ds(j * 16 + dt * 8, 8), :], sem)

    def drain(ref, sem):
        pltpu.make_async_copy(out_hbm.at[pl.ds(0, 128), :], ref, sem).wait()

    def shuffle(slab, outb):
        for ll in range(128):
            rv = rowvecs[ll // 16]
            cb = (ll % 16) * 8
            for k in range(8):
                vals = plsc.load_gather(slab, [rv, colvecs[cb + k]])
                outb[ll, pl.ds(k * 16, 16)] = vals

    @pl.when(wid == 0)
    def _tail():
        pltpu.sync_copy(tail_hbm, outb0.at[pl.ds(0, 8), :])
        pltpu.sync_copy(outb0.at[pl.ds(0, 8), :],
                        out_hbm.at[pl.ds(_NLINE - 8, 8), :])

    fire_in(wid, slab0, sem0)
    fire_in(wid + 32, slab1, sem1)

    @pl.loop(0, 40)
    def _chunks(t):
        for parity, slab, outb, sem, semo in (
                (0, slab0, outb0, sem0, semo0), (1, slab1, outb1, sem1, semo1)):
            tt = 2 * t + parity
            m = wid + 32 * tt

            @pl.when(m < _NCHUNK)
            def _():
                @pl.when(tt >= 2)
                def _():
                    drain(outb, semo)
                drain(slab, sem)
                shuffle(slab, outb)
                pltpu.async_copy(outb, out_hbm.at[pl.ds(m * 128, 128), :],
                                 semo)
                m2 = m + 64

                @pl.when(m2 < _NCHUNK)
                def _():
                    fire_in(m2, slab, sem)

    nw0 = (_NCHUNK - 1 - wid) // 32 + 1
    odd = (nw0 % 2) == 1

    @pl.when(odd)
    def _():
        drain(outb0, semo0)

    @pl.when(jnp.logical_not(odd))
    def _():
        drain(outb1, semo1)

    @pl.when((nw0 >= 2) & jnp.logical_not(odd))
    def _():
        drain(outb0, semo0)

    @pl.when((nw0 >= 2) & odd)
    def _():
        drain(outb1, semo1)


def _sc_relayout(tt, tail):
    mesh = plsc.VectorSubcoreMesh(core_axis_name="c", subcore_axis_name="s",
                                  num_cores=2, num_subcores=16)
    return pl.kernel(
        _sc_relayout_body,
        out_type=jax.ShapeDtypeStruct((_NLINE, 128), jnp.float32),
        mesh=mesh,
        scratch_types=[
            pltpu.VMEM((128, 128), jnp.float32),
            pltpu.VMEM((128, 128), jnp.float32),
            pltpu.VMEM((128, 128), jnp.float32),
            pltpu.VMEM((128, 128), jnp.float32),
            pltpu.SemaphoreType.DMA,
            pltpu.SemaphoreType.DMA,
            pltpu.SemaphoreType.DMA,
            pltpu.SemaphoreType.DMA,
        ],
        compiler_params=pltpu.CompilerParams(use_tc_tiling_on_sc=True,
                                             needs_layout_passes=False),
    )(tt, tail)


def _sc_gather_body(idx_hbm, offs_hbm, tview_hbm, out_hbm,
                    idx_v, cidx_v, offs_v, buf0, buf1, out_v, sem0, sem1):
    wid = lax.axis_index("s") * 2 + lax.axis_index("c")
    pltpu.sync_copy(idx_hbm.at[wid], idx_v)
    pltpu.sync_copy(offs_hbm, offs_v)
    for j in range(_PER_W // 16):
        r = (16 * j) // _CH
        cc = (16 * j) % _CH
        p = j % _NPAT
        v = idx_v[r, pl.ds(cc, 16)] + offs_v[p, :]
        idx_v[r, pl.ds(cc, 16)] = v
        cidx_v[r, pl.ds(cc, 16)] = lax.shift_right_logical(v, 3)

    iota16 = lax.iota(jnp.int32, 16)

    def extract(buf, k):
        for g in range(8):
            v = idx_v[k, pl.ds(g * 16, 16)]
            colbase = (v & 7) * 16
            rows16 = iota16 + g * 16
            qbase = (k * _CH + rows16) * _ED
            for l in range(16):
                vals = plsc.load_gather(buf, [rows16, colbase + l])
                q = qbase + l
                plsc.store_scatter(
                    out_v, [lax.shift_right_logical(q, 7), q & 127], vals)

    pltpu.async_copy(tview_hbm.at[cidx_v.at[0]], buf0, sem0)
    pltpu.async_copy(tview_hbm.at[cidx_v.at[1]], buf1, sem1)

    @pl.loop(0, _NCH // 2)
    def _chunks(i):
        k0 = 2 * i
        k1 = 2 * i + 1
        pltpu.make_async_copy(tview_hbm.at[cidx_v.at[k0]], buf0, sem0).wait()
        extract(buf0, k0)

        @pl.when(k0 + 2 < _NCH)
        def _():
            pltpu.async_copy(tview_hbm.at[cidx_v.at[k0 + 2]], buf0, sem0)

        pltpu.make_async_copy(tview_hbm.at[cidx_v.at[k1]], buf1, sem1).wait()
        extract(buf1, k1)

        @pl.when(k1 + 2 < _NCH)
        def _():
            pltpu.async_copy(tview_hbm.at[cidx_v.at[k1 + 2]], buf1, sem1)

    pltpu.sync_copy(out_v, out_hbm.at[wid])


def _sc_gather(idx3, offs, tview):
    mesh = plsc.VectorSubcoreMesh(core_axis_name="c", subcore_axis_name="s",
                                  num_cores=2, num_subcores=16)
    return pl.kernel(
        _sc_gather_body,
        out_type=jax.ShapeDtypeStruct((_NW, _OUTR, 128), jnp.float32),
        mesh=mesh,
        scratch_types=[
            pltpu.VMEM((_NCH, _CH), jnp.int32),
            pltpu.VMEM((_NCH, _CH), jnp.int32),
            pltpu.VMEM((_NPAT, 16), jnp.int32),
            pltpu.VMEM((_CH, 128), jnp.float32),
            pltpu.VMEM((_CH, 128), jnp.float32),
            pltpu.VMEM((_OUTR, 128), jnp.float32),
            pltpu.SemaphoreType.DMA,
            pltpu.SemaphoreType.DMA,
        ],
        compiler_params=pltpu.CompilerParams(use_tc_tiling_on_sc=True,
                                             needs_layout_passes=False),
    )(idx3, offs, tview)


def _tc_dense_body(cat_ref, nx_ref, num_w_ref, num_b_ref, ew1_ref, eb1_ref,
                   ew2_ref, eb2_ref, gw_ref, gb_ref, tw1_ref, tb1_ref,
                   tw2_ref, tb2_ref, tw3_ref, tb3_ref, out_ref):
    numem = jnp.dot(nx_ref[...], num_w_ref[...],
                    preferred_element_type=jnp.float32) + num_b_ref[...]
    emb = jnp.concatenate([cat_ref[...], numem], axis=1)
    feas = []
    for e in range(_E):
        h = jnp.maximum(
            jnp.dot(emb, ew1_ref[e], preferred_element_type=jnp.float32)
            + eb1_ref[e], 0.0)
        f = jnp.maximum(
            jnp.dot(h, ew2_ref[e], preferred_element_type=jnp.float32)
            + eb2_ref[e], 0.0)
        feas.append(f)
    outs = []
    for t in range(_T):
        g = jnp.dot(emb, gw_ref[t], preferred_element_type=jnp.float32) + gb_ref[t]
        g = jnp.exp(g - jnp.max(g, axis=1, keepdims=True))
        g = g / jnp.sum(g, axis=1, keepdims=True)
        tf = feas[0] * g[:, 0:1]
        for e in range(1, _E):
            tf = tf + feas[e] * g[:, e:e + 1]
        th = jnp.maximum(
            jnp.dot(tf, tw1_ref[t], preferred_element_type=jnp.float32)
            + tb1_ref[t], 0.0)
        th = jnp.maximum(
            jnp.dot(th, tw2_ref[t], preferred_element_type=jnp.float32)
            + tb2_ref[t], 0.0)
        o = jnp.dot(th, tw3_ref[t], preferred_element_type=jnp.float32) + tb3_ref[t]
        outs.append(1.0 / (1.0 + jnp.exp(-o)))
    out_ref[...] = jnp.concatenate(outs, axis=1)


def _tc_dense(cat_emb, numerical_x, num_w, num_b, ew1, eb1, ew2, eb2,
              gw, gb, tw1, tb1, tw2, tb2, tw3, tb3):
    def full(arr):
        nd = arr.ndim
        return pl.BlockSpec(arr.shape, lambda i, _n=nd: (0,) * _n)

    grid = (_B // _BB,)
    return pl.pallas_call(
        _tc_dense_body,
        grid=grid,
        in_specs=[
            pl.BlockSpec((_BB, _F * _ED), lambda i: (i, 0)),
            pl.BlockSpec((_BB, numerical_x.shape[1]), lambda i: (i, 0)),
            full(num_w), full(num_b), full(ew1), full(eb1), full(ew2),
            full(eb2), full(gw), full(gb), full(tw1), full(tb1), full(tw2),
            full(tb2), full(tw3), full(tb3),
        ],
        out_specs=pl.BlockSpec((_BB, _T), lambda i: (i, 0)),
        out_shape=jax.ShapeDtypeStruct((_B, _T), jnp.float32),
    )(cat_emb, numerical_x, num_w, num_b, ew1, eb1, ew2, eb2, gw, gb,
      tw1, tb1, tw2, tb2, tw3, tb3)


def kernel(categorical_x, numerical_x, embedding, num_w, num_b, ew1, eb1,
           ew2, eb2, gw, gb, tw1, tb1, tw2, tb2, tw3, tb3):
    idx3 = categorical_x.reshape(_NW, _NCH, _CH)
    offs = jnp.asarray(_offs_pattern())
    tail = embedding[(_NLINE - 8) * 8:, :].reshape(8, 128)
    tview = _sc_relayout(embedding.T, tail)
    rows = _sc_gather(idx3, offs, tview)
    cat_emb = rows.reshape(_B, _F * _ED)
    return _tc_dense(cat_emb, numerical_x, num_w, num_b, ew1, eb1, ew2, eb2,
                     gw, gb, tw1, tb1, tw2, tb2, tw3, tb3)


# speedup vs baseline: 2.7148x; 2.7148x over previous
import functools

import jax
import jax.numpy as jnp
import numpy as np
from jax import lax
from jax.experimental import pallas as pl
from jax.experimental.pallas import tpu as pltpu
from jax.experimental.pallas import tpu_sc as plsc

_FIELD_DIMS = [100000] * 26
_F = 26
_ED = 16
_B = 4096
_E = 8
_T = 2
_EMB_OUT = (_F + 1) * _ED

_NW = 32
_PER_W = _B * _F // _NW
_CH = 128
_NCH = _PER_W // _CH
_NPAT = 13
_OUTR = _PER_W * _ED // 128

_BB = 512


def _offs_pattern():
    offsets = np.concatenate([[0], np.cumsum(_FIELD_DIMS)[:-1]]).astype(np.int32)
    pat = np.array([offsets[q % _F] for q in range(_NPAT * 16)], dtype=np.int32)
    return pat.reshape(_NPAT, 16)


_NLINE = 325000
_NCHUNK = 2539
_TAILC = 64


def _sc_relayout_body(tt_hbm, tail_hbm, out_hbm, slab0, slab1, outb0, outb1,
                      sem0, sem1, semo0, semo1):
    wid = lax.axis_index("s") * 2 + lax.axis_index("c")
    iota16 = lax.iota(jnp.int32, 16)
    hi8 = lax.shift_right_logical(iota16, 3)
    lo8 = iota16 & 7

    def fire_in(m, slab, sem):
        c0 = m * 1024
        for dt in range(2):
            for j in range(8):
                pltpu.async_copy(
                    tt_hbm.at[pl.ds(dt * 8, 8), pl.ds(c0 + j * 128, 128)],
                    slab.at[pl.ds(j * 16 + dt * 8, 8), :], sem)

    def drain(ref, sem):
        pltpu.make_async_copy(out_hbm.at[pl.ds(0, 128), :], ref, sem).wait()

    def shuffle(slab, outb):
        @pl.loop(0, 128)
        def _lines(ll):
            base16 = 16 * (ll // 16)
            cv = lo8 + (ll % 16) * 8
            for g in range(8):
                rv = hi8 + (base16 + 2 * g)
                vals = plsc.load_gather(slab, [rv, cv])
                outb[ll, pl.ds(g * 16, 16)] = vals

    @pl.when(wid == 0)
    def _tail():
        pltpu.sync_copy(tail_hbm, outb0.at[pl.ds(0, 8), :])
        pltpu.sync_copy(outb0.at[pl.ds(0, 8), :],
                        out_hbm.at[pl.ds(_NLINE - 8, 8), :])

    fire_in(wid, slab0, sem0)
    fire_in(wid + 32, slab1, sem1)

    @pl.loop(0, 40)
    def _chunks(t):
        for parity, slab, outb, sem, semo in (
                (0, slab0, outb0, sem0, semo0), (1, slab1, outb1, sem1, semo1)):
            tt = 2 * t + parity
            m = wid + 32 * tt

            @pl.when(m < _NCHUNK)
            def _():
                @pl.when(tt >= 2)
                def _():
                    drain(outb, semo)
                drain(slab, sem)
                shuffle(slab, outb)
                pltpu.async_copy(outb, out_hbm.at[pl.ds(m * 128, 128), :],
                                 semo)
                m2 = m + 64

                @pl.when(m2 < _NCHUNK)
                def _():
                    fire_in(m2, slab, sem)

    nw0 = (_NCHUNK - 1 - wid) // 32 + 1
    odd = (nw0 % 2) == 1

    @pl.when(odd)
    def _():
        drain(outb0, semo0)

    @pl.when(jnp.logical_not(odd))
    def _():
        drain(outb1, semo1)

    @pl.when((nw0 >= 2) & jnp.logical_not(odd))
    def _():
        drain(outb0, semo0)

    @pl.when((nw0 >= 2) & odd)
    def _():
        drain(outb1, semo1)


def _sc_relayout(tt, tail):
    mesh = plsc.VectorSubcoreMesh(core_axis_name="c", subcore_axis_name="s",
                                  num_cores=2, num_subcores=16)
    return pl.kernel(
        _sc_relayout_body,
        out_type=jax.ShapeDtypeStruct((_NLINE, 128), jnp.float32),
        mesh=mesh,
        scratch_types=[
            pltpu.VMEM((128, 128), jnp.float32),
            pltpu.VMEM((128, 128), jnp.float32),
            pltpu.VMEM((128, 128), jnp.float32),
            pltpu.VMEM((128, 128), jnp.float32),
            pltpu.SemaphoreType.DMA,
            pltpu.SemaphoreType.DMA,
            pltpu.SemaphoreType.DMA,
            pltpu.SemaphoreType.DMA,
        ],
        compiler_params=pltpu.CompilerParams(use_tc_tiling_on_sc=True,
                                             needs_layout_passes=False),
    )(tt, tail)


def _sc_gather_body(idx_hbm, offs_hbm, tview_hbm, out_hbm,
                    idx_v, cidx_v, offs_v, buf0, buf1, out_v, sem0, sem1):
    wid = lax.axis_index("s") * 2 + lax.axis_index("c")
    pltpu.sync_copy(idx_hbm.at[wid], idx_v)
    pltpu.sync_copy(offs_hbm, offs_v)
    for j in range(_PER_W // 16):
        r = (16 * j) // _CH
        cc = (16 * j) % _CH
        p = j % _NPAT
        v = idx_v[r, pl.ds(cc, 16)] + offs_v[p, :]
        idx_v[r, pl.ds(cc, 16)] = v
        cidx_v[r, pl.ds(cc, 16)] = lax.shift_right_logical(v, 3)

    iota16 = lax.iota(jnp.int32, 16)

    def extract(buf, k):
        for g in range(8):
            v = idx_v[k, pl.ds(g * 16, 16)]
            sub = v & 7
            rows16 = iota16 + g * 16
            qbase = (k * _CH + rows16) * _ED
            for l in range(16):
                vals = plsc.load_gather(buf, [rows16, sub + 8 * l])
                q = qbase + l
                plsc.store_scatter(
                    out_v, [lax.shift_right_logical(q, 7), q & 127], vals)

    pltpu.async_copy(tview_hbm.at[cidx_v.at[0]], buf0, sem0)
    pltpu.async_copy(tview_hbm.at[cidx_v.at[1]], buf1, sem1)

    @pl.loop(0, _NCH // 2)
    def _chunks(i):
        k0 = 2 * i
        k1 = 2 * i + 1
        pltpu.make_async_copy(tview_hbm.at[cidx_v.at[k0]], buf0, sem0).wait()
        extract(buf0, k0)

        @pl.when(k0 + 2 < _NCH)
        def _():
            pltpu.async_copy(tview_hbm.at[cidx_v.at[k0 + 2]], buf0, sem0)

        pltpu.make_async_copy(tview_hbm.at[cidx_v.at[k1]], buf1, sem1).wait()
        extract(buf1, k1)

        @pl.when(k1 + 2 < _NCH)
        def _():
            pltpu.async_copy(tview_hbm.at[cidx_v.at[k1 + 2]], buf1, sem1)

    pltpu.sync_copy(out_v, out_hbm.at[wid])


def _sc_gather(idx3, offs, tview):
    mesh = plsc.VectorSubcoreMesh(core_axis_name="c", subcore_axis_name="s",
                                  num_cores=2, num_subcores=16)
    return pl.kernel(
        _sc_gather_body,
        out_type=jax.ShapeDtypeStruct((_NW, _OUTR, 128), jnp.float32),
        mesh=mesh,
        scratch_types=[
            pltpu.VMEM((_NCH, _CH), jnp.int32),
            pltpu.VMEM((_NCH, _CH), jnp.int32),
            pltpu.VMEM((_NPAT, 16), jnp.int32),
            pltpu.VMEM((_CH, 128), jnp.float32),
            pltpu.VMEM((_CH, 128), jnp.float32),
            pltpu.VMEM((_OUTR, 128), jnp.float32),
            pltpu.SemaphoreType.DMA,
            pltpu.SemaphoreType.DMA,
        ],
        compiler_params=pltpu.CompilerParams(use_tc_tiling_on_sc=True,
                                             needs_layout_passes=False),
    )(idx3, offs, tview)


def _tc_dense_body(cat_ref, nx_ref, num_w_ref, num_b_ref, ew1_ref, eb1_ref,
                   ew2_ref, eb2_ref, gw_ref, gb_ref, tw1_ref, tb1_ref,
                   tw2_ref, tb2_ref, tw3_ref, tb3_ref, out_ref):
    numem = jnp.dot(nx_ref[...], num_w_ref[...],
                    preferred_element_type=jnp.float32) + num_b_ref[...]
    emb = jnp.concatenate([cat_ref[...], numem], axis=1)
    feas = []
    for e in range(_E):
        h = jnp.maximum(
            jnp.dot(emb, ew1_ref[e], preferred_element_type=jnp.float32)
            + eb1_ref[e], 0.0)
        f = jnp.maximum(
            jnp.dot(h, ew2_ref[e], preferred_element_type=jnp.float32)
            + eb2_ref[e], 0.0)
        feas.append(f)
    outs = []
    for t in range(_T):
        g = jnp.dot(emb, gw_ref[t], preferred_element_type=jnp.float32) + gb_ref[t]
        g = jnp.exp(g - jnp.max(g, axis=1, keepdims=True))
        g = g / jnp.sum(g, axis=1, keepdims=True)
        tf = feas[0] * g[:, 0:1]
        for e in range(1, _E):
            tf = tf + feas[e] * g[:, e:e + 1]
        th = jnp.maximum(
            jnp.dot(tf, tw1_ref[t], preferred_element_type=jnp.float32)
            + tb1_ref[t], 0.0)
        th = jnp.maximum(
            jnp.dot(th, tw2_ref[t], preferred_element_type=jnp.float32)
            + tb2_ref[t], 0.0)
        o = jnp.dot(th, tw3_ref[t], preferred_element_type=jnp.float32) + tb3_ref[t]
        outs.append(1.0 / (1.0 + jnp.exp(-o)))
    out_ref[...] = jnp.concatenate(outs, axis=1)


def _tc_dense(cat_emb, numerical_x, num_w, num_b, ew1, eb1, ew2, eb2,
              gw, gb, tw1, tb1, tw2, tb2, tw3, tb3):
    def full(arr):
        nd = arr.ndim
        return pl.BlockSpec(arr.shape, lambda i, _n=nd: (0,) * _n)

    grid = (_B // _BB,)
    return pl.pallas_call(
        _tc_dense_body,
        grid=grid,
        in_specs=[
            pl.BlockSpec((_BB, _F * _ED), lambda i: (i, 0)),
            pl.BlockSpec((_BB, numerical_x.shape[1]), lambda i: (i, 0)),
            full(num_w), full(num_b), full(ew1), full(eb1), full(ew2),
            full(eb2), full(gw), full(gb), full(tw1), full(tb1), full(tw2),
            full(tb2), full(tw3), full(tb3),
        ],
        out_specs=pl.BlockSpec((_BB, _T), lambda i: (i, 0)),
        out_shape=jax.ShapeDtypeStruct((_B, _T), jnp.float32),
    )(cat_emb, numerical_x, num_w, num_b, ew1, eb1, ew2, eb2, gw, gb,
      tw1, tb1, tw2, tb2, tw3, tb3)


def kernel(categorical_x, numerical_x, embedding, num_w, num_b, ew1, eb1,
           ew2, eb2, gw, gb, tw1, tb1, tw2, tb2, tw3, tb3):
    idx3 = categorical_x.reshape(_NW, _NCH, _CH)
    offs = jnp.asarray(_offs_pattern())
    tail = embedding[(_NLINE - 8) * 8:, :].reshape(8, 8, 16)
    tail = tail.transpose(0, 2, 1).reshape(8, 128)
    tview = _sc_relayout(embedding.T, tail)
    rows = _sc_gather(idx3, offs, tview)
    cat_emb = rows.reshape(_B, _F * _ED)
    return _tc_dense(cat_emb, numerical_x, num_w, num_b, ew1, eb1, ew2, eb2,
                     gw, gb, tw1, tb1, tw2, tb2, tw3, tb3)


# speedup vs baseline: 2.7636x; 1.0180x over previous
import functools

import jax
import jax.numpy as jnp
import numpy as np
from jax import lax
from jax.experimental import pallas as pl
from jax.experimental.pallas import tpu as pltpu
from jax.experimental.pallas import tpu_sc as plsc

_FIELD_DIMS = [100000] * 26
_F = 26
_ED = 16
_B = 4096
_E = 8
_T = 2
_EMB_OUT = (_F + 1) * _ED

_NW = 32
_PER_W = _B * _F // _NW
_CH = 128
_NCH = _PER_W // _CH
_NPAT = 13
_OUTR = _PER_W * _ED // 128

_BB = 512


def _offs_pattern():
    offsets = np.concatenate([[0], np.cumsum(_FIELD_DIMS)[:-1]]).astype(np.int32)
    pat = np.array([offsets[q % _F] for q in range(_NPAT * 16)], dtype=np.int32)
    return pat.reshape(_NPAT, 16)


_NLINE = 325000
_NCHUNK = 2539
_TAILC = 64


def _sc_relayout_body(tt_hbm, tail_hbm, out_hbm, slab0, slab1, outb0, outb1,
                      sem0, sem1, semo0, semo1):
    wid = lax.axis_index("s") * 2 + lax.axis_index("c")
    iota16 = lax.iota(jnp.int32, 16)
    hi8 = lax.shift_right_logical(iota16, 3)
    lo8 = iota16 & 7

    def fire_in(m, slab, sem):
        c0 = m * 1024
        for dt in range(2):
            for j in range(8):
                pltpu.async_copy(
                    tt_hbm.at[pl.ds(dt * 8, 8), pl.ds(c0 + j * 128, 128)],
                    slab.at[pl.ds(j * 16 + dt * 8, 8), :], sem)

    def drain(ref, sem):
        pltpu.make_async_copy(out_hbm.at[pl.ds(0, 128), :], ref, sem).wait()

    def shuffle(slab, outb):
        @pl.loop(0, 128, unroll=2)
        def _lines(ll):
            base16 = 16 * (ll // 16)
            cv = lo8 + (ll % 16) * 8
            for g in range(8):
                rv = hi8 + (base16 + 2 * g)
                vals = plsc.load_gather(slab, [rv, cv])
                outb[ll, pl.ds(g * 16, 16)] = vals

    @pl.when(wid == 0)
    def _tail():
        pltpu.sync_copy(tail_hbm, outb0.at[pl.ds(0, 8), :])
        pltpu.sync_copy(outb0.at[pl.ds(0, 8), :],
                        out_hbm.at[pl.ds(_NLINE - 8, 8), :])

    fire_in(wid, slab0, sem0)
    fire_in(wid + 32, slab1, sem1)

    @pl.loop(0, 40)
    def _chunks(t):
        for parity, slab, outb, sem, semo in (
                (0, slab0, outb0, sem0, semo0), (1, slab1, outb1, sem1, semo1)):
            tt = 2 * t + parity
            m = wid + 32 * tt

            @pl.when(m < _NCHUNK)
            def _():
                @pl.when(tt >= 2)
                def _():
                    drain(outb, semo)
                drain(slab, sem)
                shuffle(slab, outb)
                pltpu.async_copy(outb, out_hbm.at[pl.ds(m * 128, 128), :],
                                 semo)
                m2 = m + 64

                @pl.when(m2 < _NCHUNK)
                def _():
                    fire_in(m2, slab, sem)

    nw0 = (_NCHUNK - 1 - wid) // 32 + 1
    odd = (nw0 % 2) == 1

    @pl.when(odd)
    def _():
        drain(outb0, semo0)

    @pl.when(jnp.logical_not(odd))
    def _():
        drain(outb1, semo1)

    @pl.when((nw0 >= 2) & jnp.logical_not(odd))
    def _():
        drain(outb0, semo0)

    @pl.when((nw0 >= 2) & odd)
    def _():
        drain(outb1, semo1)


def _sc_relayout(tt, tail):
    mesh = plsc.VectorSubcoreMesh(core_axis_name="c", subcore_axis_name="s",
                                  num_cores=2, num_subcores=16)
    return pl.kernel(
        _sc_relayout_body,
        out_type=jax.ShapeDtypeStruct((_NLINE, 128), jnp.float32),
        mesh=mesh,
        scratch_types=[
            pltpu.VMEM((128, 128), jnp.float32),
            pltpu.VMEM((128, 128), jnp.float32),
            pltpu.VMEM((128, 128), jnp.float32),
            pltpu.VMEM((128, 128), jnp.float32),
            pltpu.SemaphoreType.DMA,
            pltpu.SemaphoreType.DMA,
            pltpu.SemaphoreType.DMA,
            pltpu.SemaphoreType.DMA,
        ],
        compiler_params=pltpu.CompilerParams(use_tc_tiling_on_sc=True,
                                             needs_layout_passes=False),
    )(tt, tail)


def _sc_gather_body(idx_hbm, offs_hbm, tview_hbm, out_hbm,
                    idx_v, cidx_v, offs_v, buf0, buf1, out_v, sem0, sem1):
    wid = lax.axis_index("s") * 2 + lax.axis_index("c")
    pltpu.sync_copy(idx_hbm.at[wid], idx_v)
    pltpu.sync_copy(offs_hbm, offs_v)
    for j in range(_PER_W // 16):
        r = (16 * j) // _CH
        cc = (16 * j) % _CH
        p = j % _NPAT
        v = idx_v[r, pl.ds(cc, 16)] + offs_v[p, :]
        idx_v[r, pl.ds(cc, 16)] = v
        cidx_v[r, pl.ds(cc, 16)] = lax.shift_right_logical(v, 3)

    iota16 = lax.iota(jnp.int32, 16)

    def extract(buf, k):
        for g in range(8):
            v = idx_v[k, pl.ds(g * 16, 16)]
            sub = v & 7
            rows16 = iota16 + g * 16
            qbase = (k * _CH + rows16) * _ED
            for l in range(16):
                vals = plsc.load_gather(buf, [rows16, sub + 8 * l])
                q = qbase + l
                plsc.store_scatter(
                    out_v, [lax.shift_right_logical(q, 7), q & 127], vals)

    pltpu.async_copy(tview_hbm.at[cidx_v.at[0]], buf0, sem0)
    pltpu.async_copy(tview_hbm.at[cidx_v.at[1]], buf1, sem1)

    @pl.loop(0, _NCH // 2)
    def _chunks(i):
        k0 = 2 * i
        k1 = 2 * i + 1
        pltpu.make_async_copy(tview_hbm.at[cidx_v.at[k0]], buf0, sem0).wait()
        extract(buf0, k0)

        @pl.when(k0 + 2 < _NCH)
        def _():
            pltpu.async_copy(tview_hbm.at[cidx_v.at[k0 + 2]], buf0, sem0)

        pltpu.make_async_copy(tview_hbm.at[cidx_v.at[k1]], buf1, sem1).wait()
        extract(buf1, k1)

        @pl.when(k1 + 2 < _NCH)
        def _():
            pltpu.async_copy(tview_hbm.at[cidx_v.at[k1 + 2]], buf1, sem1)

    pltpu.sync_copy(out_v, out_hbm.at[wid])


def _sc_gather(idx3, offs, tview):
    mesh = plsc.VectorSubcoreMesh(core_axis_name="c", subcore_axis_name="s",
                                  num_cores=2, num_subcores=16)
    return pl.kernel(
        _sc_gather_body,
        out_type=jax.ShapeDtypeStruct((_NW, _OUTR, 128), jnp.float32),
        mesh=mesh,
        scratch_types=[
            pltpu.VMEM((_NCH, _CH), jnp.int32),
            pltpu.VMEM((_NCH, _CH), jnp.int32),
            pltpu.VMEM((_NPAT, 16), jnp.int32),
            pltpu.VMEM((_CH, 128), jnp.float32),
            pltpu.VMEM((_CH, 128), jnp.float32),
            pltpu.VMEM((_OUTR, 128), jnp.float32),
            pltpu.SemaphoreType.DMA,
            pltpu.SemaphoreType.DMA,
        ],
        compiler_params=pltpu.CompilerParams(use_tc_tiling_on_sc=True,
                                             needs_layout_passes=False),
    )(idx3, offs, tview)


def _tc_dense_body(cat_ref, nx_ref, num_w_ref, num_b_ref, ew1_ref, eb1_ref,
                   ew2_ref, eb2_ref, gw_ref, gb_ref, tw1_ref, tb1_ref,
                   tw2_ref, tb2_ref, tw3_ref, tb3_ref, out_ref):
    numem = jnp.dot(nx_ref[...], num_w_ref[...],
                    preferred_element_type=jnp.float32) + num_b_ref[...]
    emb = jnp.concatenate([cat_ref[...], numem], axis=1)
    emb16 = emb.astype(jnp.bfloat16)
    feas = []
    for e in range(_E):
        h = jnp.maximum(
            jnp.dot(emb16, ew1_ref[e], preferred_element_type=jnp.float32)
            + eb1_ref[e], 0.0)
        f = jnp.maximum(
            jnp.dot(h.astype(jnp.bfloat16), ew2_ref[e],
                    preferred_element_type=jnp.float32)
            + eb2_ref[e], 0.0)
        feas.append(f)
    outs = []
    for t in range(_T):
        g = jnp.dot(emb, gw_ref[t], preferred_element_type=jnp.float32) + gb_ref[t]
        g = jnp.exp(g - jnp.max(g, axis=1, keepdims=True))
        g = g / jnp.sum(g, axis=1, keepdims=True)
        tf = feas[0] * g[:, 0:1]
        for e in range(1, _E):
            tf = tf + feas[e] * g[:, e:e + 1]
        th = jnp.maximum(
            jnp.dot(tf, tw1_ref[t], preferred_element_type=jnp.float32)
            + tb1_ref[t], 0.0)
        th = jnp.maximum(
            jnp.dot(th, tw2_ref[t], preferred_element_type=jnp.float32)
            + tb2_ref[t], 0.0)
        o = jnp.dot(th, tw3_ref[t], preferred_element_type=jnp.float32) + tb3_ref[t]
        outs.append(1.0 / (1.0 + jnp.exp(-o)))
    out_ref[...] = jnp.concatenate(outs, axis=1)


def _tc_dense(cat_emb, numerical_x, num_w, num_b, ew1, eb1, ew2, eb2,
              gw, gb, tw1, tb1, tw2, tb2, tw3, tb3):
    def full(arr):
        nd = arr.ndim
        return pl.BlockSpec(arr.shape, lambda i, _n=nd: (0,) * _n)

    grid = (_B // _BB,)
    return pl.pallas_call(
        _tc_dense_body,
        grid=grid,
        in_specs=[
            pl.BlockSpec((_BB, _F * _ED), lambda i: (i, 0)),
            pl.BlockSpec((_BB, numerical_x.shape[1]), lambda i: (i, 0)),
            full(num_w), full(num_b), full(ew1), full(eb1), full(ew2),
            full(eb2), full(gw), full(gb), full(tw1), full(tb1), full(tw2),
            full(tb2), full(tw3), full(tb3),
        ],
        out_specs=pl.BlockSpec((_BB, _T), lambda i: (i, 0)),
        out_shape=jax.ShapeDtypeStruct((_B, _T), jnp.float32),
    )(cat_emb, numerical_x, num_w, num_b, ew1, eb1, ew2, eb2, gw, gb,
      tw1, tb1, tw2, tb2, tw3, tb3)


def kernel(categorical_x, numerical_x, embedding, num_w, num_b, ew1, eb1,
           ew2, eb2, gw, gb, tw1, tb1, tw2, tb2, tw3, tb3):
    idx3 = categorical_x.reshape(_NW, _NCH, _CH)
    offs = jnp.asarray(_offs_pattern())
    tail = embedding[(_NLINE - 8) * 8:, :].reshape(8, 8, 16)
    tail = tail.transpose(0, 2, 1).reshape(8, 128)
    tview = _sc_relayout(embedding.T, tail)
    rows = _sc_gather(idx3, offs, tview)
    cat_emb = rows.reshape(_B, _F * _ED)
    return _tc_dense(cat_emb, numerical_x, num_w, num_b,
                     ew1.astype(jnp.bfloat16), eb1,
                     ew2.astype(jnp.bfloat16), eb2,
                     gw, gb, tw1, tb1, tw2, tb2, tw3, tb3)


# speedup vs baseline: 2.7719x; 1.0030x over previous
import functools

import jax
import jax.numpy as jnp
import numpy as np
from jax import lax
from jax.experimental import pallas as pl
from jax.experimental.pallas import tpu as pltpu
from jax.experimental.pallas import tpu_sc as plsc

_FIELD_DIMS = [100000] * 26
_F = 26
_ED = 16
_B = 4096
_E = 8
_T = 2
_EMB_OUT = (_F + 1) * _ED

_NW = 32
_PER_W = _B * _F // _NW
_CH = 128
_NCH = _PER_W // _CH
_NPAT = 13
_OUTR = _PER_W * _ED // 128

_BB = 512


def _offs_pattern():
    offsets = np.concatenate([[0], np.cumsum(_FIELD_DIMS)[:-1]]).astype(np.int32)
    pat = np.array([offsets[q % _F] for q in range(_NPAT * 16)], dtype=np.int32)
    return pat.reshape(_NPAT, 16)


_NLINE = 325000
_NCHUNK = 2539
_TAILC = 64


def _sc_relayout_body(tt_hbm, tail_hbm, out_hbm, slab0, slab1, outb0, outb1,
                      sem0, sem1, semo0, semo1):
    wid = lax.axis_index("s") * 2 + lax.axis_index("c")
    iota16 = lax.iota(jnp.int32, 16)
    hi8 = lax.shift_right_logical(iota16, 3)
    lo8 = iota16 & 7

    zero16 = jnp.full((16,), 0, jnp.int32)
    one16 = jnp.full((16,), 1, jnp.int32)
    drvs = [hi8 + 2 * gg for gg in range(4)]

    def fire_in(m, slab, sem):
        c0 = m * 1024
        for dt in range(2):
            pltpu.async_copy(
                tt_hbm.at[pl.ds(dt * 8, 8), pl.ds(c0, 1024)],
                slab.at[dt], sem)

    def drain_in(slab, sem):
        for dt in range(2):
            pltpu.make_async_copy(
                tt_hbm.at[pl.ds(0, 8), pl.ds(0, 1024)],
                slab.at[dt], sem).wait()

    def drain(ref, sem):
        pltpu.make_async_copy(out_hbm.at[pl.ds(0, 128), :], ref, sem).wait()

    def shuffle(slab, outb):
        @pl.loop(0, 128, unroll=2)
        def _lines(ll):
            cvl = lo8 + 8 * ll
            for g in range(8):
                vals = plsc.load_gather(
                    slab, [zero16 if g < 4 else one16, drvs[g % 4], cvl])
                outb[ll, pl.ds(g * 16, 16)] = vals

    @pl.when(wid == 0)
    def _tail():
        pltpu.sync_copy(tail_hbm, outb0.at[pl.ds(0, 8), :])
        pltpu.sync_copy(outb0.at[pl.ds(0, 8), :],
                        out_hbm.at[pl.ds(_NLINE - 8, 8), :])

    fire_in(wid, slab0, sem0)
    fire_in(wid + 32, slab1, sem1)

    @pl.loop(0, 40)
    def _chunks(t):
        for parity, slab, outb, sem, semo in (
                (0, slab0, outb0, sem0, semo0), (1, slab1, outb1, sem1, semo1)):
            tt = 2 * t + parity
            m = wid + 32 * tt

            @pl.when(m < _NCHUNK)
            def _():
                @pl.when(tt >= 2)
                def _():
                    drain(outb, semo)
                drain_in(slab, sem)
                shuffle(slab, outb)
                pltpu.async_copy(outb, out_hbm.at[pl.ds(m * 128, 128), :],
                                 semo)
                m2 = m + 64

                @pl.when(m2 < _NCHUNK)
                def _():
                    fire_in(m2, slab, sem)

    nw0 = (_NCHUNK - 1 - wid) // 32 + 1
    odd = (nw0 % 2) == 1

    @pl.when(odd)
    def _():
        drain(outb0, semo0)

    @pl.when(jnp.logical_not(odd))
    def _():
        drain(outb1, semo1)

    @pl.when((nw0 >= 2) & jnp.logical_not(odd))
    def _():
        drain(outb0, semo0)

    @pl.when((nw0 >= 2) & odd)
    def _():
        drain(outb1, semo1)


def _sc_relayout(tt, tail):
    mesh = plsc.VectorSubcoreMesh(core_axis_name="c", subcore_axis_name="s",
                                  num_cores=2, num_subcores=16)
    return pl.kernel(
        _sc_relayout_body,
        out_type=jax.ShapeDtypeStruct((_NLINE, 128), jnp.float32),
        mesh=mesh,
        scratch_types=[
            pltpu.VMEM((2, 8, 1024), jnp.float32),
            pltpu.VMEM((2, 8, 1024), jnp.float32),
            pltpu.VMEM((128, 128), jnp.float32),
            pltpu.VMEM((128, 128), jnp.float32),
            pltpu.SemaphoreType.DMA,
            pltpu.SemaphoreType.DMA,
            pltpu.SemaphoreType.DMA,
            pltpu.SemaphoreType.DMA,
        ],
        compiler_params=pltpu.CompilerParams(use_tc_tiling_on_sc=True,
                                             needs_layout_passes=False),
    )(tt, tail)


def _sc_gather_body(idx_hbm, offs_hbm, tview_hbm, out_hbm,
                    idx_v, cidx_v, offs_v, buf0, buf1, out_v, sem0, sem1):
    wid = lax.axis_index("s") * 2 + lax.axis_index("c")
    pltpu.sync_copy(idx_hbm.at[wid], idx_v)
    pltpu.sync_copy(offs_hbm, offs_v)
    for j in range(_PER_W // 16):
        r = (16 * j) // _CH
        cc = (16 * j) % _CH
        p = j % _NPAT
        v = idx_v[r, pl.ds(cc, 16)] + offs_v[p, :]
        idx_v[r, pl.ds(cc, 16)] = v
        cidx_v[r, pl.ds(cc, 16)] = lax.shift_right_logical(v, 3)

    iota16 = lax.iota(jnp.int32, 16)

    def extract(buf, k):
        for g in range(8):
            v = idx_v[k, pl.ds(g * 16, 16)]
            sub = v & 7
            rows16 = iota16 + g * 16
            qbase = (k * _CH + rows16) * _ED
            for l in range(16):
                vals = plsc.load_gather(buf, [rows16, sub + 8 * l])
                q = qbase + l
                plsc.store_scatter(
                    out_v, [lax.shift_right_logical(q, 7), q & 127], vals)

    pltpu.async_copy(tview_hbm.at[cidx_v.at[0]], buf0, sem0)
    pltpu.async_copy(tview_hbm.at[cidx_v.at[1]], buf1, sem1)

    @pl.loop(0, _NCH // 2)
    def _chunks(i):
        k0 = 2 * i
        k1 = 2 * i + 1
        pltpu.make_async_copy(tview_hbm.at[cidx_v.at[k0]], buf0, sem0).wait()
        extract(buf0, k0)

        @pl.when(k0 + 2 < _NCH)
        def _():
            pltpu.async_copy(tview_hbm.at[cidx_v.at[k0 + 2]], buf0, sem0)

        pltpu.make_async_copy(tview_hbm.at[cidx_v.at[k1]], buf1, sem1).wait()
        extract(buf1, k1)

        @pl.when(k1 + 2 < _NCH)
        def _():
            pltpu.async_copy(tview_hbm.at[cidx_v.at[k1 + 2]], buf1, sem1)

    pltpu.sync_copy(out_v, out_hbm.at[wid])


def _sc_gather(idx3, offs, tview):
    mesh = plsc.VectorSubcoreMesh(core_axis_name="c", subcore_axis_name="s",
                                  num_cores=2, num_subcores=16)
    return pl.kernel(
        _sc_gather_body,
        out_type=jax.ShapeDtypeStruct((_NW, _OUTR, 128), jnp.float32),
        mesh=mesh,
        scratch_types=[
            pltpu.VMEM((_NCH, _CH), jnp.int32),
            pltpu.VMEM((_NCH, _CH), jnp.int32),
            pltpu.VMEM((_NPAT, 16), jnp.int32),
            pltpu.VMEM((_CH, 128), jnp.float32),
            pltpu.VMEM((_CH, 128), jnp.float32),
            pltpu.VMEM((_OUTR, 128), jnp.float32),
            pltpu.SemaphoreType.DMA,
            pltpu.SemaphoreType.DMA,
        ],
        compiler_params=pltpu.CompilerParams(use_tc_tiling_on_sc=True,
                                             needs_layout_passes=False),
    )(idx3, offs, tview)


def _tc_dense_body(cat_ref, nx_ref, num_w_ref, num_b_ref, ew1_ref, eb1_ref,
                   ew2_ref, eb2_ref, gw_ref, gb_ref, tw1_ref, tb1_ref,
                   tw2_ref, tb2_ref, tw3_ref, tb3_ref, out_ref):
    numem = jnp.dot(nx_ref[...], num_w_ref[...],
                    preferred_element_type=jnp.float32) + num_b_ref[...]
    emb = jnp.concatenate([cat_ref[...], numem], axis=1)
    emb16 = emb.astype(jnp.bfloat16)
    feas = []
    for e in range(_E):
        h = jnp.maximum(
            jnp.dot(emb16, ew1_ref[e], preferred_element_type=jnp.float32)
            + eb1_ref[e], 0.0)
        f = jnp.maximum(
            jnp.dot(h.astype(jnp.bfloat16), ew2_ref[e],
                    preferred_element_type=jnp.float32)
            + eb2_ref[e], 0.0)
        feas.append(f)
    outs = []
    for t in range(_T):
        g = jnp.dot(emb, gw_ref[t], preferred_element_type=jnp.float32) + gb_ref[t]
        g = jnp.exp(g - jnp.max(g, axis=1, keepdims=True))
        g = g / jnp.sum(g, axis=1, keepdims=True)
        tf = feas[0] * g[:, 0:1]
        for e in range(1, _E):
            tf = tf + feas[e] * g[:, e:e + 1]
        th = jnp.maximum(
            jnp.dot(tf, tw1_ref[t], preferred_element_type=jnp.float32)
            + tb1_ref[t], 0.0)
        th = jnp.maximum(
            jnp.dot(th, tw2_ref[t], preferred_element_type=jnp.float32)
            + tb2_ref[t], 0.0)
        o = jnp.dot(th, tw3_ref[t], preferred_element_type=jnp.float32) + tb3_ref[t]
        outs.append(1.0 / (1.0 + jnp.exp(-o)))
    out_ref[...] = jnp.concatenate(outs, axis=1)


def _tc_dense(cat_emb, numerical_x, num_w, num_b, ew1, eb1, ew2, eb2,
              gw, gb, tw1, tb1, tw2, tb2, tw3, tb3):
    def full(arr):
        nd = arr.ndim
        return pl.BlockSpec(arr.shape, lambda i, _n=nd: (0,) * _n)

    grid = (_B // _BB,)
    return pl.pallas_call(
        _tc_dense_body,
        grid=grid,
        in_specs=[
            pl.BlockSpec((_BB, _F * _ED), lambda i: (i, 0)),
            pl.BlockSpec((_BB, numerical_x.shape[1]), lambda i: (i, 0)),
            full(num_w), full(num_b), full(ew1), full(eb1), full(ew2),
            full(eb2), full(gw), full(gb), full(tw1), full(tb1), full(tw2),
            full(tb2), full(tw3), full(tb3),
        ],
        out_specs=pl.BlockSpec((_BB, _T), lambda i: (i, 0)),
        out_shape=jax.ShapeDtypeStruct((_B, _T), jnp.float32),
    )(cat_emb, numerical_x, num_w, num_b, ew1, eb1, ew2, eb2, gw, gb,
      tw1, tb1, tw2, tb2, tw3, tb3)


def kernel(categorical_x, numerical_x, embedding, num_w, num_b, ew1, eb1,
           ew2, eb2, gw, gb, tw1, tb1, tw2, tb2, tw3, tb3):
    idx3 = categorical_x.reshape(_NW, _NCH, _CH)
    offs = jnp.asarray(_offs_pattern())
    tail = embedding[(_NLINE - 8) * 8:, :].reshape(8, 8, 16)
    tail = tail.transpose(0, 2, 1).reshape(8, 128)
    tview = _sc_relayout(embedding.T, tail)
    rows = _sc_gather(idx3, offs, tview)
    cat_emb = rows.reshape(_B, _F * _ED)
    return _tc_dense(cat_emb, numerical_x, num_w, num_b,
                     ew1.astype(jnp.bfloat16), eb1,
                     ew2.astype(jnp.bfloat16), eb2,
                     gw, gb, tw1, tb1, tw2, tb2, tw3, tb3)


# speedup vs baseline: 2.8366x; 1.0233x over previous
import functools

import jax
import jax.numpy as jnp
import numpy as np
from jax import lax
from jax.experimental import pallas as pl
from jax.experimental.pallas import tpu as pltpu
from jax.experimental.pallas import tpu_sc as plsc

_FIELD_DIMS = [100000] * 26
_F = 26
_ED = 16
_B = 4096
_E = 8
_T = 2
_EMB_OUT = (_F + 1) * _ED

_NW = 32
_PER_W = _B * _F // _NW
_CH = 128
_NCH = _PER_W // _CH
_NPAT = 13
_OUTR = _PER_W * _ED // 128

_BB = 512


def _offs_pattern():
    offsets = np.concatenate([[0], np.cumsum(_FIELD_DIMS)[:-1]]).astype(np.int32)
    pat = np.array([offsets[q % _F] for q in range(_NPAT * 16)], dtype=np.int32)
    return pat.reshape(_NPAT, 16)


_NLINE = 325000
_NCHUNK = 2539
_TAILC = 64


def _sc_relayout_body(tt_hbm, tail_hbm, out_hbm, slab0, slab1, outb0, outb1,
                      sem0, sem1, semo0, semo1):
    wid = lax.axis_index("s") * 2 + lax.axis_index("c")
    iota16 = lax.iota(jnp.int32, 16)
    hi8 = lax.shift_right_logical(iota16, 3)
    lo8 = iota16 & 7

    zero16 = jnp.full((16,), 0, jnp.int32)
    one16 = jnp.full((16,), 1, jnp.int32)
    drvs = [hi8 + 2 * gg for gg in range(4)]

    def fire_in(m, slab, sem):
        c0 = m * 1024
        for dt in range(2):
            pltpu.async_copy(
                tt_hbm.at[pl.ds(dt * 8, 8), pl.ds(c0, 1024)],
                slab.at[dt], sem)

    def drain_in(slab, sem):
        for dt in range(2):
            pltpu.make_async_copy(
                tt_hbm.at[pl.ds(0, 8), pl.ds(0, 1024)],
                slab.at[dt], sem).wait()

    def drain(ref, sem):
        pltpu.make_async_copy(out_hbm.at[pl.ds(0, 128), :], ref, sem).wait()

    def shuffle(slab, outb):
        @pl.loop(0, 128, unroll=2)
        def _lines(ll):
            cvl = lo8 + 8 * ll
            for g in range(8):
                vals = plsc.load_gather(
                    slab, [zero16 if g < 4 else one16, drvs[g % 4], cvl])
                outb[ll, pl.ds(g * 16, 16)] = vals

    @pl.when(wid == 0)
    def _tail():
        pltpu.sync_copy(tail_hbm, outb0.at[pl.ds(0, 8), :])
        pltpu.sync_copy(outb0.at[pl.ds(0, 8), :],
                        out_hbm.at[pl.ds(_NLINE - 8, 8), :])

    fire_in(wid, slab0, sem0)
    fire_in(wid + 32, slab1, sem1)

    @pl.loop(0, 40)
    def _chunks(t):
        for parity, slab, outb, sem, semo in (
                (0, slab0, outb0, sem0, semo0), (1, slab1, outb1, sem1, semo1)):
            tt = 2 * t + parity
            m = wid + 32 * tt

            @pl.when(m < _NCHUNK)
            def _():
                @pl.when(tt >= 2)
                def _():
                    drain(outb, semo)
                drain_in(slab, sem)
                shuffle(slab, outb)
                pltpu.async_copy(outb, out_hbm.at[pl.ds(m * 128, 128), :],
                                 semo)
                m2 = m + 64

                @pl.when(m2 < _NCHUNK)
                def _():
                    fire_in(m2, slab, sem)

    nw0 = (_NCHUNK - 1 - wid) // 32 + 1
    odd = (nw0 % 2) == 1

    @pl.when(odd)
    def _():
        drain(outb0, semo0)

    @pl.when(jnp.logical_not(odd))
    def _():
        drain(outb1, semo1)

    @pl.when((nw0 >= 2) & jnp.logical_not(odd))
    def _():
        drain(outb0, semo0)

    @pl.when((nw0 >= 2) & odd)
    def _():
        drain(outb1, semo1)


def _sc_relayout(tt, tail):
    mesh = plsc.VectorSubcoreMesh(core_axis_name="c", subcore_axis_name="s",
                                  num_cores=2, num_subcores=16)
    return pl.kernel(
        _sc_relayout_body,
        out_type=jax.ShapeDtypeStruct((_NLINE, 128), jnp.float32),
        mesh=mesh,
        scratch_types=[
            pltpu.VMEM((2, 8, 1024), jnp.float32),
            pltpu.VMEM((2, 8, 1024), jnp.float32),
            pltpu.VMEM((128, 128), jnp.float32),
            pltpu.VMEM((128, 128), jnp.float32),
            pltpu.SemaphoreType.DMA,
            pltpu.SemaphoreType.DMA,
            pltpu.SemaphoreType.DMA,
            pltpu.SemaphoreType.DMA,
        ],
        compiler_params=pltpu.CompilerParams(use_tc_tiling_on_sc=True,
                                             needs_layout_passes=False),
    )(tt, tail)


def _sc_gather_body(idx_hbm, offs_hbm, tview_hbm, out_hbm,
                    idx_v, cidx_v, offs_v, buf0, buf1, out_v, sem0, sem1):
    nch = idx_v.shape[0]
    wid = lax.axis_index("s") * 2 + lax.axis_index("c")
    pltpu.sync_copy(idx_hbm.at[wid], idx_v)
    pltpu.sync_copy(offs_hbm, offs_v)
    for j in range(nch * _CH // 16):
        r = (16 * j) // _CH
        cc = (16 * j) % _CH
        p = j % _NPAT
        v = idx_v[r, pl.ds(cc, 16)] + offs_v[p, :]
        idx_v[r, pl.ds(cc, 16)] = v
        cidx_v[r, pl.ds(cc, 16)] = lax.shift_right_logical(v, 3)

    iota16 = lax.iota(jnp.int32, 16)

    def extract(buf, k):
        for g in range(8):
            v = idx_v[k, pl.ds(g * 16, 16)]
            sub = v & 7
            rows16 = iota16 + g * 16
            qbase = (k * _CH + rows16) * _ED
            for l in range(16):
                vals = plsc.load_gather(buf, [rows16, sub + 8 * l])
                q = qbase + l
                plsc.store_scatter(
                    out_v, [lax.shift_right_logical(q, 7), q & 127], vals)

    pltpu.async_copy(tview_hbm.at[cidx_v.at[0]], buf0, sem0)
    if nch > 1:
        pltpu.async_copy(tview_hbm.at[cidx_v.at[1]], buf1, sem1)

    @pl.loop(0, (nch + 1) // 2)
    def _chunks(i):
        k0 = 2 * i
        k1 = 2 * i + 1
        pltpu.make_async_copy(tview_hbm.at[cidx_v.at[k0]], buf0, sem0).wait()
        extract(buf0, k0)

        @pl.when(k0 + 2 < nch)
        def _():
            pltpu.async_copy(tview_hbm.at[cidx_v.at[k0 + 2]], buf0, sem0)

        @pl.when(k1 < nch)
        def _():
            pltpu.make_async_copy(
                tview_hbm.at[cidx_v.at[k1]], buf1, sem1).wait()
            extract(buf1, k1)

            @pl.when(k1 + 2 < nch)
            def _():
                pltpu.async_copy(tview_hbm.at[cidx_v.at[k1 + 2]], buf1, sem1)

    pltpu.sync_copy(out_v, out_hbm.at[wid])


def _sc_gather(idx3, offs, tview):
    nch = idx3.shape[1]
    mesh = plsc.VectorSubcoreMesh(core_axis_name="c", subcore_axis_name="s",
                                  num_cores=2, num_subcores=16)
    return pl.kernel(
        _sc_gather_body,
        out_type=jax.ShapeDtypeStruct((_NW, nch * 16, 128), jnp.float32),
        mesh=mesh,
        scratch_types=[
            pltpu.VMEM((nch, _CH), jnp.int32),
            pltpu.VMEM((nch, _CH), jnp.int32),
            pltpu.VMEM((_NPAT, 16), jnp.int32),
            pltpu.VMEM((_CH, 128), jnp.float32),
            pltpu.VMEM((_CH, 128), jnp.float32),
            pltpu.VMEM((nch * 16, 128), jnp.float32),
            pltpu.SemaphoreType.DMA,
            pltpu.SemaphoreType.DMA,
        ],
        compiler_params=pltpu.CompilerParams(use_tc_tiling_on_sc=True,
                                             needs_layout_passes=False),
    )(idx3, offs, tview)


def _tc_dense_body(cat_ref, nx_ref, num_w_ref, num_b_ref, ew1_ref, eb1_ref,
                   ew2_ref, eb2_ref, gw_ref, gb_ref, tw1_ref, tb1_ref,
                   tw2_ref, tb2_ref, tw3_ref, tb3_ref, out_ref):
    numem = jnp.dot(nx_ref[...], num_w_ref[...],
                    preferred_element_type=jnp.float32) + num_b_ref[...]
    emb = jnp.concatenate([cat_ref[...], numem], axis=1)
    emb16 = emb.astype(jnp.bfloat16)
    feas = []
    for e in range(_E):
        h = jnp.maximum(
            jnp.dot(emb16, ew1_ref[e], preferred_element_type=jnp.float32)
            + eb1_ref[e], 0.0)
        f = jnp.maximum(
            jnp.dot(h.astype(jnp.bfloat16), ew2_ref[e],
                    preferred_element_type=jnp.float32)
            + eb2_ref[e], 0.0)
        feas.append(f)
    outs = []
    for t in range(_T):
        g = jnp.dot(emb, gw_ref[t], preferred_element_type=jnp.float32) + gb_ref[t]
        g = jnp.exp(g - jnp.max(g, axis=1, keepdims=True))
        g = g / jnp.sum(g, axis=1, keepdims=True)
        tf = feas[0] * g[:, 0:1]
        for e in range(1, _E):
            tf = tf + feas[e] * g[:, e:e + 1]
        th = jnp.maximum(
            jnp.dot(tf, tw1_ref[t], preferred_element_type=jnp.float32)
            + tb1_ref[t], 0.0)
        th = jnp.maximum(
            jnp.dot(th, tw2_ref[t], preferred_element_type=jnp.float32)
            + tb2_ref[t], 0.0)
        o = jnp.dot(th, tw3_ref[t], preferred_element_type=jnp.float32) + tb3_ref[t]
        outs.append(1.0 / (1.0 + jnp.exp(-o)))
    out_ref[...] = jnp.concatenate(outs, axis=1)


def _tc_dense(cat_emb, numerical_x, num_w, num_b, ew1, eb1, ew2, eb2,
              gw, gb, tw1, tb1, tw2, tb2, tw3, tb3):
    def full(arr):
        nd = arr.ndim
        return pl.BlockSpec(arr.shape, lambda i, _n=nd: (0,) * _n)

    grid = (cat_emb.shape[0] // _BB,)
    return pl.pallas_call(
        _tc_dense_body,
        grid=grid,
        in_specs=[
            pl.BlockSpec((_BB, _F * _ED), lambda i: (i, 0)),
            pl.BlockSpec((_BB, numerical_x.shape[1]), lambda i: (i, 0)),
            full(num_w), full(num_b), full(ew1), full(eb1), full(ew2),
            full(eb2), full(gw), full(gb), full(tw1), full(tb1), full(tw2),
            full(tb2), full(tw3), full(tb3),
        ],
        out_specs=pl.BlockSpec((_BB, _T), lambda i: (i, 0)),
        out_shape=jax.ShapeDtypeStruct((cat_emb.shape[0], _T), jnp.float32),
    )(cat_emb, numerical_x, num_w, num_b, ew1, eb1, ew2, eb2, gw, gb,
      tw1, tb1, tw2, tb2, tw3, tb3)


def kernel(categorical_x, numerical_x, embedding, num_w, num_b, ew1, eb1,
           ew2, eb2, gw, gb, tw1, tb1, tw2, tb2, tw3, tb3):
    offs = jnp.asarray(_offs_pattern())
    tail = embedding[(_NLINE - 8) * 8:, :].reshape(8, 8, 16)
    tail = tail.transpose(0, 2, 1).reshape(8, 128)
    tview = _sc_relayout(embedding.T, tail)
    ew1b = ew1.astype(jnp.bfloat16)
    ew2b = ew2.astype(jnp.bfloat16)
    half = _B // 2
    outs = []
    for h in range(2):
        cat_h = categorical_x[h * half:(h + 1) * half]
        idx3 = cat_h.reshape(_NW, half * _F // (_NW * _CH), _CH)
        rows = _sc_gather(idx3, offs, tview)
        cat_emb = rows.reshape(half, _F * _ED)
        outs.append(_tc_dense(
            cat_emb, numerical_x[h * half:(h + 1) * half], num_w, num_b,
            ew1b, eb1, ew2b, eb2, gw, gb, tw1, tb1, tw2, tb2, tw3, tb3))
    return jnp.concatenate(outs, axis=0)


# speedup vs baseline: 2.8580x; 1.0076x over previous
import functools

import jax
import jax.numpy as jnp
import numpy as np
from jax import lax
from jax.experimental import pallas as pl
from jax.experimental.pallas import tpu as pltpu
from jax.experimental.pallas import tpu_sc as plsc

_FIELD_DIMS = [100000] * 26
_F = 26
_ED = 16
_B = 4096
_E = 8
_T = 2
_EMB_OUT = (_F + 1) * _ED

_NW = 32
_PER_W = _B * _F // _NW
_CH = 128
_NCH = _PER_W // _CH
_NPAT = 13
_OUTR = _PER_W * _ED // 128

_BB = 1024


def _offs_pattern():
    offsets = np.concatenate([[0], np.cumsum(_FIELD_DIMS)[:-1]]).astype(np.int32)
    pat = np.array([offsets[q % _F] for q in range(_NPAT * 16)], dtype=np.int32)
    return pat.reshape(_NPAT, 16)


_NLINE = 325000
_NCHUNK = 2539
_TAILC = 64


def _sc_relayout_body(tt_hbm, tail_hbm, out_hbm, slab0, slab1, outb0, outb1,
                      sem0, sem1, semo0, semo1):
    wid = lax.axis_index("s") * 2 + lax.axis_index("c")
    iota16 = lax.iota(jnp.int32, 16)
    hi8 = lax.shift_right_logical(iota16, 3)
    lo8 = iota16 & 7

    zero16 = jnp.full((16,), 0, jnp.int32)
    one16 = jnp.full((16,), 1, jnp.int32)
    drvs = [hi8 + 2 * gg for gg in range(4)]

    def fire_in(m, slab, sem):
        c0 = m * 1024
        for dt in range(2):
            pltpu.async_copy(
                tt_hbm.at[pl.ds(dt * 8, 8), pl.ds(c0, 1024)],
                slab.at[dt], sem)

    def drain_in(slab, sem):
        for dt in range(2):
            pltpu.make_async_copy(
                tt_hbm.at[pl.ds(0, 8), pl.ds(0, 1024)],
                slab.at[dt], sem).wait()

    def drain(ref, sem):
        pltpu.make_async_copy(out_hbm.at[pl.ds(0, 128), :], ref, sem).wait()

    def shuffle(slab, outb):
        @pl.loop(0, 128, unroll=2)
        def _lines(ll):
            cvl = lo8 + 8 * ll
            for g in range(8):
                vals = plsc.load_gather(
                    slab, [zero16 if g < 4 else one16, drvs[g % 4], cvl])
                outb[ll, pl.ds(g * 16, 16)] = vals

    @pl.when(wid == 0)
    def _tail():
        pltpu.sync_copy(tail_hbm, outb0.at[pl.ds(0, 8), :])
        pltpu.sync_copy(outb0.at[pl.ds(0, 8), :],
                        out_hbm.at[pl.ds(_NLINE - 8, 8), :])

    fire_in(wid, slab0, sem0)
    fire_in(wid + 32, slab1, sem1)

    @pl.loop(0, 40)
    def _chunks(t):
        for parity, slab, outb, sem, semo in (
                (0, slab0, outb0, sem0, semo0), (1, slab1, outb1, sem1, semo1)):
            tt = 2 * t + parity
            m = wid + 32 * tt

            @pl.when(m < _NCHUNK)
            def _():
                @pl.when(tt >= 2)
                def _():
                    drain(outb, semo)
                drain_in(slab, sem)
                shuffle(slab, outb)
                pltpu.async_copy(outb, out_hbm.at[pl.ds(m * 128, 128), :],
                                 semo)
                m2 = m + 64

                @pl.when(m2 < _NCHUNK)
                def _():
                    fire_in(m2, slab, sem)

    nw0 = (_NCHUNK - 1 - wid) // 32 + 1
    odd = (nw0 % 2) == 1

    @pl.when(odd)
    def _():
        drain(outb0, semo0)

    @pl.when(jnp.logical_not(odd))
    def _():
        drain(outb1, semo1)

    @pl.when((nw0 >= 2) & jnp.logical_not(odd))
    def _():
        drain(outb0, semo0)

    @pl.when((nw0 >= 2) & odd)
    def _():
        drain(outb1, semo1)


def _sc_relayout(tt, tail):
    mesh = plsc.VectorSubcoreMesh(core_axis_name="c", subcore_axis_name="s",
                                  num_cores=2, num_subcores=16)
    return pl.kernel(
        _sc_relayout_body,
        out_type=jax.ShapeDtypeStruct((_NLINE, 128), jnp.float32),
        mesh=mesh,
        scratch_types=[
            pltpu.VMEM((2, 8, 1024), jnp.float32),
            pltpu.VMEM((2, 8, 1024), jnp.float32),
            pltpu.VMEM((128, 128), jnp.float32),
            pltpu.VMEM((128, 128), jnp.float32),
            pltpu.SemaphoreType.DMA,
            pltpu.SemaphoreType.DMA,
            pltpu.SemaphoreType.DMA,
            pltpu.SemaphoreType.DMA,
        ],
        compiler_params=pltpu.CompilerParams(use_tc_tiling_on_sc=True,
                                             needs_layout_passes=False),
    )(tt, tail)


def _sc_gather_body(idx_hbm, offs_hbm, tview_hbm, out_hbm,
                    idx_v, cidx_v, offs_v, buf0, buf1, out_v, sem0, sem1):
    nch = idx_v.shape[0]
    wid = lax.axis_index("s") * 2 + lax.axis_index("c")
    pltpu.sync_copy(idx_hbm.at[wid], idx_v)
    pltpu.sync_copy(offs_hbm, offs_v)
    for j in range(nch * _CH // 16):
        r = (16 * j) // _CH
        cc = (16 * j) % _CH
        p = j % _NPAT
        v = idx_v[r, pl.ds(cc, 16)] + offs_v[p, :]
        idx_v[r, pl.ds(cc, 16)] = v
        cidx_v[r, pl.ds(cc, 16)] = lax.shift_right_logical(v, 3)

    iota16 = lax.iota(jnp.int32, 16)

    def extract(buf, k):
        for g in range(8):
            v = idx_v[k, pl.ds(g * 16, 16)]
            sub = v & 7
            rows16 = iota16 + g * 16
            qbase = (k * _CH + rows16) * _ED
            for l in range(16):
                vals = plsc.load_gather(buf, [rows16, sub + 8 * l])
                q = qbase + l
                plsc.store_scatter(
                    out_v, [lax.shift_right_logical(q, 7), q & 127], vals)

    pltpu.async_copy(tview_hbm.at[cidx_v.at[0]], buf0, sem0)
    if nch > 1:
        pltpu.async_copy(tview_hbm.at[cidx_v.at[1]], buf1, sem1)

    @pl.loop(0, (nch + 1) // 2)
    def _chunks(i):
        k0 = 2 * i
        k1 = 2 * i + 1
        pltpu.make_async_copy(tview_hbm.at[cidx_v.at[k0]], buf0, sem0).wait()
        extract(buf0, k0)

        @pl.when(k0 + 2 < nch)
        def _():
            pltpu.async_copy(tview_hbm.at[cidx_v.at[k0 + 2]], buf0, sem0)

        @pl.when(k1 < nch)
        def _():
            pltpu.make_async_copy(
                tview_hbm.at[cidx_v.at[k1]], buf1, sem1).wait()
            extract(buf1, k1)

            @pl.when(k1 + 2 < nch)
            def _():
                pltpu.async_copy(tview_hbm.at[cidx_v.at[k1 + 2]], buf1, sem1)

    pltpu.sync_copy(out_v, out_hbm.at[wid])


def _sc_gather(idx3, offs, tview):
    nch = idx3.shape[1]
    mesh = plsc.VectorSubcoreMesh(core_axis_name="c", subcore_axis_name="s",
                                  num_cores=2, num_subcores=16)
    return pl.kernel(
        _sc_gather_body,
        out_type=jax.ShapeDtypeStruct((_NW, nch * 16, 128), jnp.float32),
        mesh=mesh,
        scratch_types=[
            pltpu.VMEM((nch, _CH), jnp.int32),
            pltpu.VMEM((nch, _CH), jnp.int32),
            pltpu.VMEM((_NPAT, 16), jnp.int32),
            pltpu.VMEM((_CH, 128), jnp.float32),
            pltpu.VMEM((_CH, 128), jnp.float32),
            pltpu.VMEM((nch * 16, 128), jnp.float32),
            pltpu.SemaphoreType.DMA,
            pltpu.SemaphoreType.DMA,
        ],
        compiler_params=pltpu.CompilerParams(use_tc_tiling_on_sc=True,
                                             needs_layout_passes=False),
    )(idx3, offs, tview)


def _tc_dense_body(cat_ref, nx_ref, num_w_ref, num_b_ref, ew1_ref, eb1_ref,
                   ew2_ref, eb2_ref, gw_ref, gb_ref, tw1_ref, tb1_ref,
                   tw2_ref, tb2_ref, tw3_ref, tb3_ref, out_ref):
    numem = jnp.dot(nx_ref[...], num_w_ref[...],
                    preferred_element_type=jnp.float32) + num_b_ref[...]
    emb = jnp.concatenate([cat_ref[...], numem], axis=1)
    emb16 = emb.astype(jnp.bfloat16)
    feas = []
    for e in range(_E):
        h = jnp.maximum(
            jnp.dot(emb16, ew1_ref[e], preferred_element_type=jnp.float32)
            + eb1_ref[e], 0.0)
        f = jnp.maximum(
            jnp.dot(h.astype(jnp.bfloat16), ew2_ref[e],
                    preferred_element_type=jnp.float32)
            + eb2_ref[e], 0.0)
        feas.append(f)
    outs = []
    for t in range(_T):
        g = jnp.dot(emb, gw_ref[t], preferred_element_type=jnp.float32) + gb_ref[t]
        g = jnp.exp(g - jnp.max(g, axis=1, keepdims=True))
        g = g / jnp.sum(g, axis=1, keepdims=True)
        tf = feas[0] * g[:, 0:1]
        for e in range(1, _E):
            tf = tf + feas[e] * g[:, e:e + 1]
        th = jnp.maximum(
            jnp.dot(tf, tw1_ref[t], preferred_element_type=jnp.float32)
            + tb1_ref[t], 0.0)
        th = jnp.maximum(
            jnp.dot(th, tw2_ref[t], preferred_element_type=jnp.float32)
            + tb2_ref[t], 0.0)
        o = jnp.dot(th, tw3_ref[t], preferred_element_type=jnp.float32) + tb3_ref[t]
        outs.append(1.0 / (1.0 + jnp.exp(-o)))
    out_ref[...] = jnp.concatenate(outs, axis=1)


def _tc_dense(cat_emb, numerical_x, num_w, num_b, ew1, eb1, ew2, eb2,
              gw, gb, tw1, tb1, tw2, tb2, tw3, tb3):
    def full(arr):
        nd = arr.ndim
        return pl.BlockSpec(arr.shape, lambda i, _n=nd: (0,) * _n)

    grid = (cat_emb.shape[0] // _BB,)
    return pl.pallas_call(
        _tc_dense_body,
        grid=grid,
        in_specs=[
            pl.BlockSpec((_BB, _F * _ED), lambda i: (i, 0)),
            pl.BlockSpec((_BB, numerical_x.shape[1]), lambda i: (i, 0)),
            full(num_w), full(num_b), full(ew1), full(eb1), full(ew2),
            full(eb2), full(gw), full(gb), full(tw1), full(tb1), full(tw2),
            full(tb2), full(tw3), full(tb3),
        ],
        out_specs=pl.BlockSpec((_BB, _T), lambda i: (i, 0)),
        out_shape=jax.ShapeDtypeStruct((cat_emb.shape[0], _T), jnp.float32),
    )(cat_emb, numerical_x, num_w, num_b, ew1, eb1, ew2, eb2, gw, gb,
      tw1, tb1, tw2, tb2, tw3, tb3)


def kernel(categorical_x, numerical_x, embedding, num_w, num_b, ew1, eb1,
           ew2, eb2, gw, gb, tw1, tb1, tw2, tb2, tw3, tb3):
    offs = jnp.asarray(_offs_pattern())
    tail = embedding[(_NLINE - 8) * 8:, :].reshape(8, 8, 16)
    tail = tail.transpose(0, 2, 1).reshape(8, 128)
    tview = _sc_relayout(embedding.T, tail)
    ew1b = ew1.astype(jnp.bfloat16)
    ew2b = ew2.astype(jnp.bfloat16)
    half = _B // 2
    outs = []
    for h in range(2):
        cat_h = categorical_x[h * half:(h + 1) * half]
        idx3 = cat_h.reshape(_NW, half * _F // (_NW * _CH), _CH)
        rows = _sc_gather(idx3, offs, tview)
        cat_emb = rows.reshape(half, _F * _ED)
        outs.append(_tc_dense(
            cat_emb, numerical_x[h * half:(h + 1) * half], num_w, num_b,
            ew1b, eb1, ew2b, eb2, gw, gb, tw1, tb1, tw2, tb2, tw3, tb3))
    return jnp.concatenate(outs, axis=0)


# speedup vs baseline: 2.8590x; 1.0004x over previous
import jax
import jax.numpy as jnp
import numpy as np
from jax import lax
from jax.experimental import pallas as pl
from jax.experimental.pallas import tpu as pltpu
from jax.experimental.pallas import tpu_sc as plsc

_FIELD_DIMS = [100000] * 26
_F = 26
_ED = 16
_B = 4096
_E = 8
_T = 2
_NW = 32
_CH = 128
_NPAT = 13

_BB = 1024


def _offs_pattern():
    offsets = np.concatenate([[0], np.cumsum(_FIELD_DIMS)[:-1]]).astype(np.int32)
    pat = np.array([offsets[q % _F] for q in range(_NPAT * 16)], dtype=np.int32)
    return pat.reshape(_NPAT, 16)


_NLINE = 325000
_NCHUNK = 2539
_TAILC = 64


def _sc_relayout_body(tt_hbm, tail_hbm, out_hbm, slab0, slab1, outb0, outb1,
                      sem0, sem1, semo0, semo1):
    wid = lax.axis_index("s") * 2 + lax.axis_index("c")
    iota16 = lax.iota(jnp.int32, 16)
    hi8 = lax.shift_right_logical(iota16, 3)
    lo8 = iota16 & 7

    zero16 = jnp.full((16,), 0, jnp.int32)
    one16 = jnp.full((16,), 1, jnp.int32)
    drvs = [hi8 + 2 * gg for gg in range(4)]

    def fire_in(m, slab, sem):
        c0 = m * 1024
        for dt in range(2):
            pltpu.async_copy(
                tt_hbm.at[pl.ds(dt * 8, 8), pl.ds(c0, 1024)],
                slab.at[dt], sem)

    def drain_in(slab, sem):
        for dt in range(2):
            pltpu.make_async_copy(
                tt_hbm.at[pl.ds(0, 8), pl.ds(0, 1024)],
                slab.at[dt], sem).wait()

    def drain(ref, sem):
        pltpu.make_async_copy(out_hbm.at[pl.ds(0, 128), :], ref, sem).wait()

    def shuffle(slab, outb):
        @pl.loop(0, 128, unroll=2)
        def _lines(ll):
            cvl = lo8 + 8 * ll
            for g in range(8):
                vals = plsc.load_gather(
                    slab, [zero16 if g < 4 else one16, drvs[g % 4], cvl])
                outb[ll, pl.ds(g * 16, 16)] = vals

    @pl.when(wid == 0)
    def _tail():
        pltpu.sync_copy(tail_hbm, outb0.at[pl.ds(0, 8), :])
        pltpu.sync_copy(outb0.at[pl.ds(0, 8), :],
                        out_hbm.at[pl.ds(_NLINE - 8, 8), :])

    fire_in(wid, slab0, sem0)
    fire_in(wid + 32, slab1, sem1)

    @pl.loop(0, 40)
    def _chunks(t):
        for parity, slab, outb, sem, semo in (
                (0, slab0, outb0, sem0, semo0), (1, slab1, outb1, sem1, semo1)):
            tt = 2 * t + parity
            m = wid + 32 * tt

            @pl.when(m < _NCHUNK)
            def _():
                @pl.when(tt >= 2)
                def _():
                    drain(outb, semo)
                drain_in(slab, sem)
                shuffle(slab, outb)
                pltpu.async_copy(outb, out_hbm.at[pl.ds(m * 128, 128), :],
                                 semo)
                m2 = m + 64

                @pl.when(m2 < _NCHUNK)
                def _():
                    fire_in(m2, slab, sem)

    nw0 = (_NCHUNK - 1 - wid) // 32 + 1
    odd = (nw0 % 2) == 1

    @pl.when(odd)
    def _():
        drain(outb0, semo0)

    @pl.when(jnp.logical_not(odd))
    def _():
        drain(outb1, semo1)

    @pl.when((nw0 >= 2) & jnp.logical_not(odd))
    def _():
        drain(outb0, semo0)

    @pl.when((nw0 >= 2) & odd)
    def _():
        drain(outb1, semo1)


def _sc_relayout(tt, tail):
    mesh = plsc.VectorSubcoreMesh(core_axis_name="c", subcore_axis_name="s",
                                  num_cores=2, num_subcores=16)
    return pl.kernel(
        _sc_relayout_body,
        out_type=jax.ShapeDtypeStruct((_NLINE, 128), jnp.float32),
        mesh=mesh,
        scratch_types=[
            pltpu.VMEM((2, 8, 1024), jnp.float32),
            pltpu.VMEM((2, 8, 1024), jnp.float32),
            pltpu.VMEM((128, 128), jnp.float32),
            pltpu.VMEM((128, 128), jnp.float32),
            pltpu.SemaphoreType.DMA,
            pltpu.SemaphoreType.DMA,
            pltpu.SemaphoreType.DMA,
            pltpu.SemaphoreType.DMA,
        ],
        compiler_params=pltpu.CompilerParams(use_tc_tiling_on_sc=True,
                                             needs_layout_passes=False),
    )(tt, tail)


def _sc_gather_body(idx_hbm, offs_hbm, tview_hbm, out_hbm,
                    idx_v, cidx_v, offs_v, buf0, buf1, out_v, sem0, sem1):
    nch = idx_v.shape[0]
    wid = lax.axis_index("s") * 2 + lax.axis_index("c")
    pltpu.sync_copy(idx_hbm.at[wid], idx_v)
    pltpu.sync_copy(offs_hbm, offs_v)
    for j in range(nch * _CH // 16):
        r = (16 * j) // _CH
        cc = (16 * j) % _CH
        p = j % _NPAT
        v = idx_v[r, pl.ds(cc, 16)] + offs_v[p, :]
        idx_v[r, pl.ds(cc, 16)] = v
        cidx_v[r, pl.ds(cc, 16)] = lax.shift_right_logical(v, 3)

    iota16 = lax.iota(jnp.int32, 16)

    def extract(buf, k):
        for g in range(8):
            v = idx_v[k, pl.ds(g * 16, 16)]
            sub = v & 7
            rows16 = iota16 + g * 16
            qbase = (k * _CH + rows16) * _ED
            for l in range(16):
                vals = plsc.load_gather(buf, [rows16, sub + 8 * l])
                q = qbase + l
                plsc.store_scatter(
                    out_v, [lax.shift_right_logical(q, 7), q & 127], vals)

    pltpu.async_copy(tview_hbm.at[cidx_v.at[0]], buf0, sem0)
    if nch > 1:
        pltpu.async_copy(tview_hbm.at[cidx_v.at[1]], buf1, sem1)

    @pl.loop(0, (nch + 1) // 2)
    def _chunks(i):
        k0 = 2 * i
        k1 = 2 * i + 1
        pltpu.make_async_copy(tview_hbm.at[cidx_v.at[k0]], buf0, sem0).wait()
        extract(buf0, k0)

        @pl.when(k0 + 2 < nch)
        def _():
            pltpu.async_copy(tview_hbm.at[cidx_v.at[k0 + 2]], buf0, sem0)

        @pl.when(k1 < nch)
        def _():
            pltpu.make_async_copy(
                tview_hbm.at[cidx_v.at[k1]], buf1, sem1).wait()
            extract(buf1, k1)

            @pl.when(k1 + 2 < nch)
            def _():
                pltpu.async_copy(tview_hbm.at[cidx_v.at[k1 + 2]], buf1, sem1)

    pltpu.sync_copy(out_v, out_hbm.at[wid])


def _sc_gather(idx3, offs, tview):
    nch = idx3.shape[1]
    mesh = plsc.VectorSubcoreMesh(core_axis_name="c", subcore_axis_name="s",
                                  num_cores=2, num_subcores=16)
    return pl.kernel(
        _sc_gather_body,
        out_type=jax.ShapeDtypeStruct((_NW, nch * 16, 128), jnp.float32),
        mesh=mesh,
        scratch_types=[
            pltpu.VMEM((nch, _CH), jnp.int32),
            pltpu.VMEM((nch, _CH), jnp.int32),
            pltpu.VMEM((_NPAT, 16), jnp.int32),
            pltpu.VMEM((_CH, 128), jnp.float32),
            pltpu.VMEM((_CH, 128), jnp.float32),
            pltpu.VMEM((nch * 16, 128), jnp.float32),
            pltpu.SemaphoreType.DMA,
            pltpu.SemaphoreType.DMA,
        ],
        compiler_params=pltpu.CompilerParams(use_tc_tiling_on_sc=True,
                                             needs_layout_passes=False),
    )(idx3, offs, tview)


def _tc_dense_body(cat_ref, nx_ref, num_w_ref, num_b_ref, ew1_ref, eb1_ref,
                   ew2_ref, eb2_ref, gw_ref, gb_ref, tw1_ref, tb1_ref,
                   tw2_ref, tb2_ref, tw3_ref, tb3_ref, out_ref):
    numem = jnp.dot(nx_ref[...], num_w_ref[...],
                    preferred_element_type=jnp.float32) + num_b_ref[...]
    emb = jnp.concatenate([cat_ref[...], numem], axis=1)
    emb16 = emb.astype(jnp.bfloat16)
    feas = []
    for e in range(_E):
        h = jnp.maximum(
            jnp.dot(emb16, ew1_ref[e], preferred_element_type=jnp.float32)
            + eb1_ref[e], 0.0)
        f = jnp.maximum(
            jnp.dot(h.astype(jnp.bfloat16), ew2_ref[e],
                    preferred_element_type=jnp.float32)
            + eb2_ref[e], 0.0)
        feas.append(f)
    outs = []
    for t in range(_T):
        g = jnp.dot(emb, gw_ref[t], preferred_element_type=jnp.float32) + gb_ref[t]
        g = jnp.exp(g - jnp.max(g, axis=1, keepdims=True))
        g = g / jnp.sum(g, axis=1, keepdims=True)
        tf = feas[0] * g[:, 0:1]
        for e in range(1, _E):
            tf = tf + feas[e] * g[:, e:e + 1]
        th = jnp.maximum(
            jnp.dot(tf, tw1_ref[t], preferred_element_type=jnp.float32)
            + tb1_ref[t], 0.0)
        th = jnp.maximum(
            jnp.dot(th, tw2_ref[t], preferred_element_type=jnp.float32)
            + tb2_ref[t], 0.0)
        o = jnp.dot(th, tw3_ref[t], preferred_element_type=jnp.float32) + tb3_ref[t]
        outs.append(1.0 / (1.0 + jnp.exp(-o)))
    out_ref[...] = jnp.concatenate(outs, axis=1)


def _tc_dense(cat_emb, numerical_x, num_w, num_b, ew1, eb1, ew2, eb2,
              gw, gb, tw1, tb1, tw2, tb2, tw3, tb3):
    def full(arr):
        nd = arr.ndim
        return pl.BlockSpec(arr.shape, lambda i, _n=nd: (0,) * _n)

    grid = (cat_emb.shape[0] // _BB,)
    return pl.pallas_call(
        _tc_dense_body,
        grid=grid,
        in_specs=[
            pl.BlockSpec((_BB, _F * _ED), lambda i: (i, 0)),
            pl.BlockSpec((_BB, numerical_x.shape[1]), lambda i: (i, 0)),
            full(num_w), full(num_b), full(ew1), full(eb1), full(ew2),
            full(eb2), full(gw), full(gb), full(tw1), full(tb1), full(tw2),
            full(tb2), full(tw3), full(tb3),
        ],
        out_specs=pl.BlockSpec((_BB, _T), lambda i: (i, 0)),
        out_shape=jax.ShapeDtypeStruct((cat_emb.shape[0], _T), jnp.float32),
    )(cat_emb, numerical_x, num_w, num_b, ew1, eb1, ew2, eb2, gw, gb,
      tw1, tb1, tw2, tb2, tw3, tb3)


def kernel(categorical_x, numerical_x, embedding, num_w, num_b, ew1, eb1,
           ew2, eb2, gw, gb, tw1, tb1, tw2, tb2, tw3, tb3):
    offs = jnp.asarray(_offs_pattern())
    tail = embedding[(_NLINE - 8) * 8:, :].reshape(8, 8, 16)
    tail = tail.transpose(0, 2, 1).reshape(8, 128)
    tview = _sc_relayout(embedding.T, tail)
    ew1b = ew1.astype(jnp.bfloat16)
    ew2b = ew2.astype(jnp.bfloat16)
    half = _B // 2
    outs = []
    for h in range(2):
        cat_h = categorical_x[h * half:(h + 1) * half]
        idx3 = cat_h.reshape(_NW, half * _F // (_NW * _CH), _CH)
        rows = _sc_gather(idx3, offs, tview)
        cat_emb = rows.reshape(half, _F * _ED)
        outs.append(_tc_dense(
            cat_emb, numerical_x[h * half:(h + 1) * half], num_w, num_b,
            ew1b, eb1, ew2b, eb2, gw, gb, tw1, tb1, tw2, tb2, tw3, tb3))
    return jnp.concatenate(outs, axis=0)
